# Initial kernel scaffold; baseline (speedup 1.0000x reference)
import jax
import jax.numpy as jnp
from jax.experimental import pallas as pl


def kernel(X, edge_index, intevrals, spans, Wc, a_src, a_dst, bc, W_ti, b_ti, W_ti1, b_ti1, W_ti2, W_ts, b_ts, W_ts1, b_ts1, b_gates):
    raise NotImplementedError("write your pallas kernel here")


# speedup vs baseline: 48.9779x; 48.9779x over previous
import functools

import jax
import jax.numpy as jnp
from jax import lax
from jax.experimental import pallas as pl
from jax.experimental.pallas import tpu as pltpu

N = 10000
D = 128
E = 160000
ET = E + N
EC = 512
NB = 80
NBLK = N // NB
EPAD = ((ET + 2 * EC - 1) // EC) * EC
LIVE = (1, 2, 4, 5)


def _prep_body(x_ref, iv_ref, sp_ref, wc_ref, asrc_ref, adst_ref, bc_ref,
               wti_ref, bti_ref, wti2_ref, wts_ref, bts_ref, wts1_ref,
               bts1_ref, bg_ref, ad_ref, u_ref, addc_ref):
    x = x_ref[...]
    ucols, vcols = [], []
    for c in LIVE:
        for h in range(2):
            wch = wc_ref[c, :, h * D:(h + 1) * D]
            ucols.append(jnp.dot(wch, asrc_ref[c, h][:, None],
                                 preferred_element_type=jnp.float32))
            vcols.append(jnp.dot(wch, adst_ref[c, h][:, None],
                                 preferred_element_type=jnp.float32))
    zpad = jnp.zeros((D, 8), dtype=jnp.float32)
    u = jnp.concatenate(ucols + [zpad], axis=1)
    v = jnp.concatenate(vcols + [zpad], axis=1)
    u_ref[...] = u
    ad_ref[...] = jnp.dot(x, v, preferred_element_type=jnp.float32)
    tmean = jnp.mean(jnp.tanh(iv_ref[...] * wti_ref[...] + bti_ref[...]),
                     axis=0, keepdims=True)
    smean = jnp.mean(jnp.tanh(sp_ref[...] * wts_ref[...] + bts_ref[...]),
                     axis=0, keepdims=True)
    r_ts = (bc_ref[1][None] + bts1_ref[...]
            + jnp.dot(smean, wts1_ref[...], preferred_element_type=jnp.float32))
    r_i = bc_ref[2][None] + bg_ref[0][None]
    r_c = bc_ref[4][None] + bg_ref[2][None]
    r_o = (bc_ref[5][None] + bg_ref[3][None]
           + 2.0 * jnp.dot(tmean, wti2_ref[...], preferred_element_type=jnp.float32))
    zrow = jnp.zeros((4, D), dtype=jnp.float32)
    addc_ref[...] = jnp.concatenate([r_ts, r_i, r_c, r_o, zrow], axis=0)


def _dense_prep(X, intevrals, spans, Wc, a_src, a_dst, bc,
                W_ti, b_ti, W_ti2, W_ts, b_ts, W_ts1, b_ts1, b_gates):
    return pl.pallas_call(
        _prep_body,
        out_shape=[
            jax.ShapeDtypeStruct((N, 16), jnp.float32),
            jax.ShapeDtypeStruct((D, 16), jnp.float32),
            jax.ShapeDtypeStruct((8, D), jnp.float32),
        ],
    )(X, intevrals[:, None], spans[:, None], Wc, a_src, a_dst, bc,
      W_ti, b_ti, W_ti2, W_ts, b_ts, W_ts1, b_ts1, b_gates)


def _fused_body(bstart_ref, xg_ref, d2_ref, u_ref, ad_ref, wstk_ref, addc_ref,
                h_ref, c_ref, xg_v, d2_v, acc, den, sem1, sem2):
    b = pl.program_id(0)
    n0 = b * NB
    win_lo = bstart_ref[b]
    win_hi = bstart_ref[b + 1]
    astart = (win_lo // 8) * 8
    nc = (win_hi - astart + EC - 1) // EC

    acc[...] = jnp.zeros((8, NB, D), dtype=jnp.float32)
    den[...] = jnp.zeros((NB, 16), dtype=jnp.float32)

    u = u_ref[...]
    adblk = ad_ref[...]

    def body(i, _):
        off = astart + i * EC
        cp1 = pltpu.make_async_copy(xg_ref.at[pl.ds(off, EC), :], xg_v, sem1)
        cp2 = pltpu.make_async_copy(d2_ref.at[pl.ds(off, EC), :], d2_v, sem2)
        cp1.start()
        cp2.start()
        cp1.wait()
        cp2.wait()
        xgc = xg_v[...]
        d2c = d2_v[...]
        ids = n0 + lax.broadcasted_iota(jnp.int32, (EC, NB), 1)
        indt = (d2c == ids).astype(jnp.float32)
        asrc = jnp.dot(xgc, u, preferred_element_type=jnp.float32)
        adst = jnp.dot(indt, adblk, preferred_element_type=jnp.float32)
        lg = asrc + adst
        w = jnp.exp(jnp.where(lg >= 0.0, lg, 0.2 * lg))
        den[...] += lax.dot_general(indt, w, (((0,), (0,)), ((), ())),
                                    preferred_element_type=jnp.float32)
        for k in range(8):
            mk = indt * w[:, k:k + 1]
            acc[k] += lax.dot_general(mk, xgc, (((0,), (0,)), ((), ())),
                                      preferred_element_type=jnp.float32)
        return 0

    lax.fori_loop(0, nc, body, 0)

    dg = den[...] + 1e-16
    gates = []
    for ci in range(4):
        a0 = acc[2 * ci] / dg[:, 2 * ci:2 * ci + 1]
        a1 = acc[2 * ci + 1] / dg[:, 2 * ci + 1:2 * ci + 2]
        g = 0.5 * (jnp.dot(a0, wstk_ref[2 * ci], preferred_element_type=jnp.float32)
                   + jnp.dot(a1, wstk_ref[2 * ci + 1], preferred_element_type=jnp.float32))
        gates.append(g + addc_ref[ci][None])
    si = jax.nn.sigmoid(gates[0])
    gi = jax.nn.sigmoid(gates[1])
    gt = jnp.tanh(gates[2])
    go = jax.nn.sigmoid(gates[3])
    cc = gi * si * gt
    c_ref[...] = cc
    h_ref[...] = go * jnp.tanh(cc)


def _fused_call(bstart, Xg, d2s, U, AD, Wstk, addc):
    grid_spec = pltpu.PrefetchScalarGridSpec(
        num_scalar_prefetch=1,
        grid=(NBLK,),
        in_specs=[
            pl.BlockSpec(memory_space=pl.ANY),
            pl.BlockSpec(memory_space=pl.ANY),
            pl.BlockSpec((D, 16), lambda b, s: (0, 0)),
            pl.BlockSpec((NB, 16), lambda b, s: (b, 0)),
            pl.BlockSpec((8, D, D), lambda b, s: (0, 0, 0)),
            pl.BlockSpec((8, D), lambda b, s: (0, 0)),
        ],
        out_specs=[
            pl.BlockSpec((NB, D), lambda b, s: (b, 0)),
            pl.BlockSpec((NB, D), lambda b, s: (b, 0)),
        ],
        scratch_shapes=[
            pltpu.VMEM((EC, D), jnp.float32),
            pltpu.VMEM((EC, 1), jnp.int32),
            pltpu.VMEM((8, NB, D), jnp.float32),
            pltpu.VMEM((NB, 16), jnp.float32),
            pltpu.SemaphoreType.DMA,
            pltpu.SemaphoreType.DMA,
        ],
    )
    return pl.pallas_call(
        _fused_body,
        grid_spec=grid_spec,
        out_shape=[
            jax.ShapeDtypeStruct((N, D), jnp.float32),
            jax.ShapeDtypeStruct((N, D), jnp.float32),
        ],
    )(bstart, Xg, d2s, U, AD, Wstk, addc)


def kernel(X, edge_index, intevrals, spans, Wc, a_src, a_dst, bc,
           W_ti, b_ti, W_ti1, b_ti1, W_ti2, W_ts, b_ts, W_ts1, b_ts1, b_gates):
    src = edge_index[0].astype(jnp.int32)
    dst = edge_index[1].astype(jnp.int32)
    loop = jnp.arange(N, dtype=jnp.int32)
    s2 = jnp.concatenate([src, loop])
    d2 = jnp.concatenate([dst, loop])
    key = d2 * 32768 + s2
    key = jnp.concatenate([key, jnp.full((EPAD - ET,), N * 32768, jnp.int32)])
    key = jnp.sort(key)
    d2s = key // 32768
    s2s = key - d2s * 32768
    bstart = jnp.searchsorted(
        d2s, jnp.arange(0, N + NB, NB, dtype=jnp.int32)).astype(jnp.int32)

    Xg = X[s2s]

    Wstk = jnp.stack([Wc[c, :, h * D:(h + 1) * D] for c in LIVE for h in range(2)])
    AD, U, addc = _dense_prep(X, intevrals, spans, Wc, a_src, a_dst, bc,
                              W_ti, b_ti, W_ti2, W_ts, b_ts, W_ts1, b_ts1, b_gates)
    H, C = _fused_call(bstart, Xg, d2s[:, None], U, AD, Wstk, addc)
    return (H, C)


# speedup vs baseline: 62.4171x; 1.2744x over previous
import functools

import jax
import jax.numpy as jnp
from jax import lax
from jax.experimental import pallas as pl
from jax.experimental.pallas import tpu as pltpu
from jax.experimental.pallas import tpu_sc as plsc

N = 10000
D = 128
E = 160000
ET = E + N
EC = 512
NB = 80
NBLK = N // NB
NWORK = 32
GCH = 128
GSTEPS = 42
EPAD = NWORK * GCH * GSTEPS
LIVE = (1, 2, 4, 5)


def _prep_body(x_ref, iv_ref, sp_ref, wc_ref, asrc_ref, adst_ref, bc_ref,
               wti_ref, bti_ref, wti2_ref, wts_ref, bts_ref, wts1_ref,
               bts1_ref, bg_ref, ad_ref, u_ref, addc_ref):
    x = x_ref[...]
    ucols, vcols = [], []
    for c in LIVE:
        for h in range(2):
            wch = wc_ref[c, :, h * D:(h + 1) * D]
            ucols.append(jnp.dot(wch, asrc_ref[c, h][:, None],
                                 preferred_element_type=jnp.float32))
            vcols.append(jnp.dot(wch, adst_ref[c, h][:, None],
                                 preferred_element_type=jnp.float32))
    zpad = jnp.zeros((D, 8), dtype=jnp.float32)
    u = jnp.concatenate(ucols + [zpad], axis=1)
    v = jnp.concatenate(vcols + [zpad], axis=1)
    u_ref[...] = u
    ad_ref[...] = jnp.dot(x, v, preferred_element_type=jnp.float32)
    tmean = jnp.mean(jnp.tanh(iv_ref[...] * wti_ref[...] + bti_ref[...]),
                     axis=0, keepdims=True)
    smean = jnp.mean(jnp.tanh(sp_ref[...] * wts_ref[...] + bts_ref[...]),
                     axis=0, keepdims=True)
    r_ts = (bc_ref[1][None] + bts1_ref[...]
            + jnp.dot(smean, wts1_ref[...], preferred_element_type=jnp.float32))
    r_i = bc_ref[2][None] + bg_ref[0][None]
    r_c = bc_ref[4][None] + bg_ref[2][None]
    r_o = (bc_ref[5][None] + bg_ref[3][None]
           + 2.0 * jnp.dot(tmean, wti2_ref[...], preferred_element_type=jnp.float32))
    zrow = jnp.zeros((4, D), dtype=jnp.float32)
    addc_ref[...] = jnp.concatenate([r_ts, r_i, r_c, r_o, zrow], axis=0)


def _dense_prep(X, intevrals, spans, Wc, a_src, a_dst, bc,
                W_ti, b_ti, W_ti2, W_ts, b_ts, W_ts1, b_ts1, b_gates):
    return pl.pallas_call(
        _prep_body,
        out_shape=[
            jax.ShapeDtypeStruct((N, 16), jnp.float32),
            jax.ShapeDtypeStruct((D, 16), jnp.float32),
            jax.ShapeDtypeStruct((8, D), jnp.float32),
        ],
    )(X, intevrals[:, None], spans[:, None], Wc, a_src, a_dst, bc,
      W_ti, b_ti, W_ti2, W_ts, b_ts, W_ts1, b_ts1, b_gates)


def _gather_body(x_hbm, idx_hbm, out_hbm, idx_v, rows_v, sem):
    wid = lax.axis_index("c") * 16 + lax.axis_index("s")
    base = wid * (GCH * GSTEPS)

    def step(j, _):
        off = base + j * GCH
        pltpu.sync_copy(idx_hbm.at[pl.ds(off, GCH)], idx_v)
        pltpu.async_copy(x_hbm.at[idx_v], rows_v, sem).wait()
        pltpu.sync_copy(rows_v, out_hbm.at[pl.ds(off, GCH), :])
        return 0

    lax.fori_loop(0, GSTEPS, step, 0)


def _sc_gather(X, s2s):
    mesh = plsc.VectorSubcoreMesh(core_axis_name="c", subcore_axis_name="s")
    return pl.kernel(
        _gather_body,
        mesh=mesh,
        out_type=jax.ShapeDtypeStruct((EPAD, D), jnp.float32),
        scratch_types=[
            pltpu.VMEM((GCH,), jnp.int32),
            pltpu.VMEM((GCH, D), jnp.float32),
            pltpu.SemaphoreType.DMA,
        ],
    )(X, s2s)


def _fused_body(bstart_ref, xg_ref, d2_ref, u_ref, ad_ref, wstk_ref, addc_ref,
                h_ref, c_ref, xg_v, d2_v, acc, den, sem1, sem2):
    b = pl.program_id(0)
    n0 = b * NB
    win_lo = bstart_ref[b]
    win_hi = bstart_ref[b + 1]
    astart = (win_lo // 8) * 8
    nc = (win_hi - astart + EC - 1) // EC

    acc[...] = jnp.zeros((8, NB, D), dtype=jnp.float32)
    den[...] = jnp.zeros((NB, 16), dtype=jnp.float32)

    u = u_ref[...]
    adblk = ad_ref[...]

    def body(i, _):
        off = astart + i * EC
        cp1 = pltpu.make_async_copy(xg_ref.at[pl.ds(off, EC), :], xg_v, sem1)
        cp2 = pltpu.make_async_copy(d2_ref.at[pl.ds(off, EC), :], d2_v, sem2)
        cp1.start()
        cp2.start()
        cp1.wait()
        cp2.wait()
        xgc = xg_v[...]
        d2c = d2_v[...]
        ids = n0 + lax.broadcasted_iota(jnp.int32, (EC, NB), 1)
        indt = (d2c == ids).astype(jnp.float32)
        asrc = jnp.dot(xgc, u, preferred_element_type=jnp.float32)
        adst = jnp.dot(indt, adblk, preferred_element_type=jnp.float32)
        lg = asrc + adst
        w = jnp.exp(jnp.where(lg >= 0.0, lg, 0.2 * lg))
        den[...] += lax.dot_general(indt, w, (((0,), (0,)), ((), ())),
                                    preferred_element_type=jnp.float32)
        for k in range(8):
            mk = indt * w[:, k:k + 1]
            acc[k] += lax.dot_general(mk, xgc, (((0,), (0,)), ((), ())),
                                      preferred_element_type=jnp.float32)
        return 0

    lax.fori_loop(0, nc, body, 0)

    dg = den[...] + 1e-16
    gates = []
    for ci in range(4):
        a0 = acc[2 * ci] / dg[:, 2 * ci:2 * ci + 1]
        a1 = acc[2 * ci + 1] / dg[:, 2 * ci + 1:2 * ci + 2]
        g = 0.5 * (jnp.dot(a0, wstk_ref[2 * ci], preferred_element_type=jnp.float32)
                   + jnp.dot(a1, wstk_ref[2 * ci + 1], preferred_element_type=jnp.float32))
        gates.append(g + addc_ref[ci][None])
    si = jax.nn.sigmoid(gates[0])
    gi = jax.nn.sigmoid(gates[1])
    gt = jnp.tanh(gates[2])
    go = jax.nn.sigmoid(gates[3])
    cc = gi * si * gt
    c_ref[...] = cc
    h_ref[...] = go * jnp.tanh(cc)


def _fused_call(bstart, Xg, d2s, U, AD, Wstk, addc):
    grid_spec = pltpu.PrefetchScalarGridSpec(
        num_scalar_prefetch=1,
        grid=(NBLK,),
        in_specs=[
            pl.BlockSpec(memory_space=pl.ANY),
            pl.BlockSpec(memory_space=pl.ANY),
            pl.BlockSpec((D, 16), lambda b, s: (0, 0)),
            pl.BlockSpec((NB, 16), lambda b, s: (b, 0)),
            pl.BlockSpec((8, D, D), lambda b, s: (0, 0, 0)),
            pl.BlockSpec((8, D), lambda b, s: (0, 0)),
        ],
        out_specs=[
            pl.BlockSpec((NB, D), lambda b, s: (b, 0)),
            pl.BlockSpec((NB, D), lambda b, s: (b, 0)),
        ],
        scratch_shapes=[
            pltpu.VMEM((EC, D), jnp.float32),
            pltpu.VMEM((EC, 1), jnp.int32),
            pltpu.VMEM((8, NB, D), jnp.float32),
            pltpu.VMEM((NB, 16), jnp.float32),
            pltpu.SemaphoreType.DMA,
            pltpu.SemaphoreType.DMA,
        ],
    )
    return pl.pallas_call(
        _fused_body,
        grid_spec=grid_spec,
        out_shape=[
            jax.ShapeDtypeStruct((N, D), jnp.float32),
            jax.ShapeDtypeStruct((N, D), jnp.float32),
        ],
    )(bstart, Xg, d2s, U, AD, Wstk, addc)


def kernel(X, edge_index, intevrals, spans, Wc, a_src, a_dst, bc,
           W_ti, b_ti, W_ti1, b_ti1, W_ti2, W_ts, b_ts, W_ts1, b_ts1, b_gates):
    src = edge_index[0].astype(jnp.int32)
    dst = edge_index[1].astype(jnp.int32)
    loop = jnp.arange(N, dtype=jnp.int32)
    s2 = jnp.concatenate([src, loop])
    d2 = jnp.concatenate([dst, loop])
    key = d2 * 32768 + s2
    key = jnp.concatenate([key, jnp.full((EPAD - ET,), N * 32768, jnp.int32)])
    key = jnp.sort(key)
    d2s = key // 32768
    s2s = key - d2s * 32768
    bstart = jnp.searchsorted(
        d2s, jnp.arange(0, N + NB, NB, dtype=jnp.int32)).astype(jnp.int32)

    Xg = _sc_gather(X, s2s)

    Wstk = jnp.stack([Wc[c, :, h * D:(h + 1) * D] for c in LIVE for h in range(2)])
    AD, U, addc = _dense_prep(X, intevrals, spans, Wc, a_src, a_dst, bc,
                              W_ti, b_ti, W_ti2, W_ts, b_ts, W_ts1, b_ts1, b_gates)
    H, C = _fused_call(bstart, Xg, d2s[:, None], U, AD, Wstk, addc)
    return (H, C)
